# Initial kernel scaffold; baseline (speedup 1.0000x reference)
import jax
import jax.numpy as jnp
from jax.experimental import pallas as pl


def kernel(hidden_states, gate_w, e_score_correction_bias, gate_up_proj, down_proj, shared_gate_w, shared_up_w, shared_down_w):
    raise NotImplementedError("write your pallas kernel here")


# speedup vs baseline: 1.9562x; 1.9562x over previous
import functools

import jax
import jax.numpy as jnp
from jax.experimental import pallas as pl
from jax.experimental.pallas import tpu as pltpu

T = 2048
H = 1024
I = 512
E = 16
SI = 1024
TOPK = 2
SCALE = 2.5
CHUNK = 512

NEG_INF = -1e30


def _routing_we(x, gate_w, bias, e):
    logits = jax.lax.dot_general(
        x, gate_w, (((1,), (1,)), ((), ())),
        preferred_element_type=jnp.float32)
    scores = jax.nn.sigmoid(logits)
    sc = scores + bias
    iota = jax.lax.broadcasted_iota(jnp.int32, (T, E), 1)
    m1 = jnp.max(sc, axis=-1, keepdims=True)
    idx1 = jnp.min(jnp.where(sc == m1, iota, E), axis=-1, keepdims=True)
    sc2 = jnp.where(iota == idx1, NEG_INF, sc)
    m2 = jnp.max(sc2, axis=-1, keepdims=True)
    idx2 = jnp.min(jnp.where(sc2 == m2, iota, E), axis=-1, keepdims=True)
    w1 = jnp.sum(jnp.where(iota == idx1, scores, 0.0), axis=-1, keepdims=True)
    w2 = jnp.sum(jnp.where(iota == idx2, scores, 0.0), axis=-1, keepdims=True)
    denom = w1 + w2 + 1e-20
    w1 = w1 / denom * SCALE
    w2 = w2 / denom * SCALE
    we = jnp.where(idx1 == e, w1, 0.0) + jnp.where(idx2 == e, w2, 0.0)
    return we


def _moe_kernel(x_ref, gate_w_ref, bias_ref, gup_ref, down_ref,
                sgw_ref, suw_ref, sdw_ref, out_ref):
    e = pl.program_id(0)
    x = x_ref[...]
    we = _routing_we(x, gate_w_ref[...], bias_ref[...], e)

    gup = gup_ref[0]
    dwn = down_ref[0]

    @pl.when(e == 0)
    def _init():
        for c in range(T // CHUNK):
            xc = x[c * CHUNK:(c + 1) * CHUNK]
            sg = jax.lax.dot_general(xc, sgw_ref[...], (((1,), (1,)), ((), ())),
                                     preferred_element_type=jnp.float32)
            su = jax.lax.dot_general(xc, suw_ref[...], (((1,), (1,)), ((), ())),
                                     preferred_element_type=jnp.float32)
            h = jax.nn.silu(sg) * su
            y = jax.lax.dot_general(h, sdw_ref[...], (((1,), (1,)), ((), ())),
                                    preferred_element_type=jnp.float32)
            out_ref[c * CHUNK:(c + 1) * CHUNK, :] = y

    for c in range(T // CHUNK):
        xc = x[c * CHUNK:(c + 1) * CHUNK]
        gu = jax.lax.dot_general(xc, gup, (((1,), (1,)), ((), ())),
                                 preferred_element_type=jnp.float32)
        h = jax.nn.silu(gu[:, :I]) * gu[:, I:]
        y = jax.lax.dot_general(h, dwn, (((1,), (1,)), ((), ())),
                                preferred_element_type=jnp.float32)
        out_ref[c * CHUNK:(c + 1) * CHUNK, :] += \
            we[c * CHUNK:(c + 1) * CHUNK] * y


def kernel(hidden_states, gate_w, e_score_correction_bias, gate_up_proj,
           down_proj, shared_gate_w, shared_up_w, shared_down_w):
    x = hidden_states.reshape(T, H)
    bias = e_score_correction_bias.reshape(1, E)

    out = pl.pallas_call(
        _moe_kernel,
        grid=(E,),
        in_specs=[
            pl.BlockSpec((T, H), lambda e: (0, 0)),
            pl.BlockSpec((E, H), lambda e: (0, 0)),
            pl.BlockSpec((1, E), lambda e: (0, 0)),
            pl.BlockSpec((1, 2 * I, H), lambda e: (e, 0, 0)),
            pl.BlockSpec((1, H, I), lambda e: (e, 0, 0)),
            pl.BlockSpec((SI, H), lambda e: (0, 0)),
            pl.BlockSpec((SI, H), lambda e: (0, 0)),
            pl.BlockSpec((H, SI), lambda e: (0, 0)),
        ],
        out_specs=pl.BlockSpec((T, H), lambda e: (0, 0)),
        out_shape=jax.ShapeDtypeStruct((T, H), jnp.float32),
        compiler_params=pltpu.CompilerParams(
            dimension_semantics=("arbitrary",),
        ),
    )(x, gate_w, bias, gate_up_proj, down_proj,
      shared_gate_w, shared_up_w, shared_down_w)

    return out.reshape(hidden_states.shape)
